# baseline (speedup 1.0000x reference)
import functools

import jax
import jax.numpy as jnp
from jax import lax
from jax.experimental import pallas as pl
from jax.experimental.pallas import tpu as pltpu
from jax.experimental.pallas import tpu_sc as plsc

B = 4096
L = 200
EMBED = 64
HIDDEN = 128

NUM_CORES = 2
NUM_SUBCORES = 16
NW = NUM_CORES * NUM_SUBCORES
SENT_PER_W = B // NW
CHUNK0 = 128
CHUNK1 = L - CHUNK0


def _gather_sums_body(x_hbm, table_hbm, sums_hbm, idx_v, rows_v, acc_v, sem):
    wid = lax.axis_index("s") * NUM_CORES + lax.axis_index("c")
    base = wid * SENT_PER_W
    pltpu.sync_copy(x_hbm.at[pl.ds(base * L, SENT_PER_W * L)], idx_v)

    def sent_body(s, carry):
        off = pl.multiple_of(s * L, 8)
        c1 = pltpu.async_copy(
            table_hbm.at[idx_v.at[pl.ds(off, CHUNK0)]],
            rows_v.at[pl.ds(0, CHUNK0)], sem)
        c2 = pltpu.async_copy(
            table_hbm.at[idx_v.at[pl.ds(off + CHUNK0, CHUNK1)]],
            rows_v.at[pl.ds(CHUNK0, CHUNK1)], sem)
        c1.wait()
        c2.wait()

        def red(r, accs):
            return tuple(accs[c] + rows_v[r, pl.ds(c * 16, 16)]
                         for c in range(EMBED // 16))

        z = jnp.zeros((16,), jnp.float32)
        accs = lax.fori_loop(0, L, red, (z, z, z, z))
        for c in range(EMBED // 16):
            acc_v[s, pl.ds(c * 16, 16)] = accs[c]
        return carry

    lax.fori_loop(0, SENT_PER_W, sent_body, 0)
    pltpu.sync_copy(acc_v, sums_hbm.at[pl.ds(base, SENT_PER_W)])


_gather_sums = functools.partial(
    pl.kernel,
    out_type=jax.ShapeDtypeStruct((B, EMBED), jnp.float32),
    mesh=plsc.VectorSubcoreMesh(core_axis_name="c", subcore_axis_name="s",
                                num_cores=NUM_CORES,
                                num_subcores=NUM_SUBCORES),
    compiler_params=pltpu.CompilerParams(use_tc_tiling_on_sc=False),
    scratch_types=[
        pltpu.VMEM((SENT_PER_W * L,), jnp.int32),
        pltpu.VMEM((L, EMBED), jnp.float32),
        pltpu.VMEM((SENT_PER_W, EMBED), jnp.float32),
        pltpu.SemaphoreType.DMA,
    ],
)(_gather_sums_body)


def _mlp_body(x_ref, sums_ref, w1_ref, b1_ref, w2_ref, b2_ref, out_ref):
    cnt = jnp.sum((x_ref[...] != 0).astype(jnp.float32), axis=1, keepdims=True)
    pooled = sums_ref[...] / (cnt + 1e-8)
    h = jnp.maximum(
        jnp.dot(pooled, w1_ref[...], preferred_element_type=jnp.float32)
        + b1_ref[...], 0.0)
    out_ref[...] = (
        jnp.dot(h, w2_ref[...], preferred_element_type=jnp.float32)
        + b2_ref[...])


BLK = 512


def kernel(x, table, W1, b1, W2, b2):
    sums = _gather_sums(x.reshape(-1), table)
    out = pl.pallas_call(
        _mlp_body,
        grid=(B // BLK,),
        in_specs=[
            pl.BlockSpec((BLK, L), lambda i: (i, 0)),
            pl.BlockSpec((BLK, EMBED), lambda i: (i, 0)),
            pl.BlockSpec((EMBED, HIDDEN), lambda i: (0, 0)),
            pl.BlockSpec((1, HIDDEN), lambda i: (0, 0)),
            pl.BlockSpec((HIDDEN, EMBED), lambda i: (0, 0)),
            pl.BlockSpec((1, EMBED), lambda i: (0, 0)),
        ],
        out_specs=pl.BlockSpec((BLK, EMBED), lambda i: (i, 0)),
        out_shape=jax.ShapeDtypeStruct((B, EMBED), jnp.float32),
    )(x, sums, W1, b1.reshape(1, HIDDEN), W2, b2.reshape(1, EMBED))
    return out


# speedup vs baseline: 1.2359x; 1.2359x over previous
import functools

import jax
import jax.numpy as jnp
from jax import lax
from jax.experimental import pallas as pl
from jax.experimental.pallas import tpu as pltpu
from jax.experimental.pallas import tpu_sc as plsc

B = 4096
L = 200
EMBED = 64
HIDDEN = 128

NUM_CORES = 2
NUM_SUBCORES = 16
NW = NUM_CORES * NUM_SUBCORES
SENT_PER_W = B // NW
CHUNK0 = 128
CHUNK1 = L - CHUNK0


NBUF = 4
RUNROLL = 8


def _gather_sums_body(x_hbm, table_hbm, sums_hbm, idx_v, rows_v, acc_v,
                      sem0, sem1, sem2, sem3):
    sems = (sem0, sem1, sem2, sem3)
    wid = lax.axis_index("s") * NUM_CORES + lax.axis_index("c")
    base = wid * SENT_PER_W
    pltpu.sync_copy(x_hbm.at[pl.ds(base * L, SENT_PER_W * L)], idx_v)

    def fire(s, b):
        off = pl.multiple_of(s * L, 8)
        pltpu.async_copy(
            table_hbm.at[idx_v.at[pl.ds(off, CHUNK0)]],
            rows_v.at[b, pl.ds(0, CHUNK0)], sems[b])
        pltpu.async_copy(
            table_hbm.at[idx_v.at[pl.ds(off + CHUNK0, CHUNK1)]],
            rows_v.at[b, pl.ds(CHUNK0, CHUNK1)], sems[b])

    def wait(b):
        pltpu.make_async_copy(
            table_hbm.at[idx_v.at[pl.ds(0, CHUNK0)]],
            rows_v.at[b, pl.ds(0, CHUNK0)], sems[b]).wait()
        pltpu.make_async_copy(
            table_hbm.at[idx_v.at[pl.ds(0, CHUNK1)]],
            rows_v.at[b, pl.ds(CHUNK0, CHUNK1)], sems[b]).wait()

    def reduce(s, b):
        def red(t, accs):
            r0 = t * RUNROLL
            for dr in range(RUNROLL):
                accs = tuple(accs[c] + rows_v[b, r0 + dr, pl.ds(c * 16, 16)]
                             for c in range(EMBED // 16))
            return accs

        z = jnp.zeros((16,), jnp.float32)
        accs = lax.fori_loop(0, L // RUNROLL, red, (z, z, z, z))
        for c in range(EMBED // 16):
            acc_v[s, pl.ds(c * 16, 16)] = accs[c]

    for b in range(NBUF):
        fire(b, b)

    def group(g, carry):
        s0 = g * NBUF
        for b in range(NBUF):
            wait(b)
            reduce(s0 + b, b)
            fire(s0 + b + NBUF, b)
        return carry

    lax.fori_loop(0, SENT_PER_W // NBUF - 1, group, 0)
    s0 = SENT_PER_W - NBUF
    for b in range(NBUF):
        wait(b)
        reduce(s0 + b, b)

    pltpu.sync_copy(acc_v, sums_hbm.at[pl.ds(base, SENT_PER_W)])


_gather_sums = functools.partial(
    pl.kernel,
    out_type=jax.ShapeDtypeStruct((B, EMBED), jnp.float32),
    mesh=plsc.VectorSubcoreMesh(core_axis_name="c", subcore_axis_name="s",
                                num_cores=NUM_CORES,
                                num_subcores=NUM_SUBCORES),
    compiler_params=pltpu.CompilerParams(use_tc_tiling_on_sc=False),
    scratch_types=[
        pltpu.VMEM((SENT_PER_W * L,), jnp.int32),
        pltpu.VMEM((NBUF, L, EMBED), jnp.float32),
        pltpu.VMEM((SENT_PER_W, EMBED), jnp.float32),
        pltpu.SemaphoreType.DMA,
        pltpu.SemaphoreType.DMA,
        pltpu.SemaphoreType.DMA,
        pltpu.SemaphoreType.DMA,
    ],
)(_gather_sums_body)


def _mlp_body(x_ref, sums_ref, w1_ref, b1_ref, w2_ref, b2_ref, out_ref):
    cnt = jnp.sum((x_ref[...] != 0).astype(jnp.float32), axis=1, keepdims=True)
    pooled = sums_ref[...] / (cnt + 1e-8)
    h = jnp.maximum(
        jnp.dot(pooled, w1_ref[...], preferred_element_type=jnp.float32)
        + b1_ref[...], 0.0)
    out_ref[...] = (
        jnp.dot(h, w2_ref[...], preferred_element_type=jnp.float32)
        + b2_ref[...])


BLK = 512


def kernel(x, table, W1, b1, W2, b2):
    sums = _gather_sums(x.reshape(-1), table)
    out = pl.pallas_call(
        _mlp_body,
        grid=(B // BLK,),
        in_specs=[
            pl.BlockSpec((BLK, L), lambda i: (i, 0)),
            pl.BlockSpec((BLK, EMBED), lambda i: (i, 0)),
            pl.BlockSpec((EMBED, HIDDEN), lambda i: (0, 0)),
            pl.BlockSpec((1, HIDDEN), lambda i: (0, 0)),
            pl.BlockSpec((HIDDEN, EMBED), lambda i: (0, 0)),
            pl.BlockSpec((1, EMBED), lambda i: (0, 0)),
        ],
        out_specs=pl.BlockSpec((BLK, EMBED), lambda i: (i, 0)),
        out_shape=jax.ShapeDtypeStruct((B, EMBED), jnp.float32),
    )(x, sums, W1, b1.reshape(1, HIDDEN), W2, b2.reshape(1, EMBED))
    return out
